# Initial kernel scaffold; baseline (speedup 1.0000x reference)
import jax
import jax.numpy as jnp
from jax.experimental import pallas as pl


def kernel(x, edge_index, batch, global_features, params):
    raise NotImplementedError("write your pallas kernel here")


# speedup vs baseline: 9.9372x; 9.9372x over previous
import functools

import jax
import jax.numpy as jnp
from jax import lax
from jax.experimental import pallas as pl
from jax.experimental.pallas import tpu as pltpu
from jax.experimental.pallas import tpu_sc as plsc

N = 10000
E = 320000
H = 128
G = 64
NL = 4

NC = 2
NS = 16
NW = NC * NS
EW = E // NW
CHUNK = 80
NCHUNK = EW // CHUNK
NPAD = 10240
ZROWS = NPAD // NS

_mesh = plsc.VectorSubcoreMesh(core_axis_name="c", subcore_axis_name="s")


def _fill_1d(ref, n, value):
    for j in range(n // 16):
        ref[pl.ds(16 * j, 16)] = jnp.full((16,), value, jnp.float32)


def _sc_degree_body(dst_hbm, out_hbm, didx, ones, zbuf, deg_acc, sem):
    c = lax.axis_index("c")
    s = lax.axis_index("s")
    wid = s * NC + c

    _fill_1d(ones, CHUNK, 1.0)
    _fill_1d(zbuf, ZROWS, 0.0)
    pltpu.sync_copy(zbuf, deg_acc.at[pl.ds(ZROWS * s, ZROWS)])
    plsc.subcore_barrier()

    def body(i, _):
        base = wid * EW + CHUNK * i
        pltpu.sync_copy(dst_hbm.at[pl.ds(base, CHUNK)], didx)
        pltpu.sync_copy(ones, deg_acc.at[didx], add=True)
        return ()

    lax.fori_loop(0, NCHUNK, body, (), unroll=False)
    plsc.subcore_barrier()
    pltpu.sync_copy(deg_acc.at[pl.ds(ZROWS * s, ZROWS)],
                    out_hbm.at[c, pl.ds(ZROWS * s, ZROWS)])


_sc_degree = functools.partial(
    pl.kernel,
    out_type=jax.ShapeDtypeStruct((NC, NPAD), jnp.float32),
    mesh=_mesh,
    scratch_types=[
        pltpu.VMEM((CHUNK,), jnp.int32),
        pltpu.VMEM((CHUNK,), jnp.float32),
        pltpu.VMEM((ZROWS,), jnp.float32),
        pltpu.VMEM_SHARED((NPAD,), jnp.float32),
        pltpu.SemaphoreType.DMA,
    ],
)(_sc_degree_body)


def _sc_agg_body(hp_hbm, src_hbm, dst_hbm, out_hbm,
                 sidx, didx, rows, zb, acc, sem):
    c = lax.axis_index("c")
    s = lax.axis_index("s")
    wid = s * NC + c

    for i in range(16):
        for j in range(8):
            zb[i, pl.ds(16 * j, 16)] = jnp.zeros((16,), jnp.float32)

    def zbody(k, _):
        pltpu.sync_copy(zb, acc.at[pl.ds(ZROWS * s + 16 * k, 16)])
        return ()

    lax.fori_loop(0, ZROWS // 16, zbody, (), unroll=False)
    plsc.subcore_barrier()

    def body(i, _):
        base = wid * EW + CHUNK * i
        pltpu.sync_copy(src_hbm.at[pl.ds(base, CHUNK)], sidx)
        pltpu.sync_copy(dst_hbm.at[pl.ds(base, CHUNK)], didx)
        pltpu.async_copy(hp_hbm.at[sidx], rows, sem).wait()
        pltpu.sync_copy(rows, acc.at[didx], add=True)
        return ()

    lax.fori_loop(0, NCHUNK, body, (), unroll=False)
    plsc.subcore_barrier()
    pltpu.sync_copy(acc.at[pl.ds(ZROWS * s, ZROWS)],
                    out_hbm.at[c, pl.ds(ZROWS * s, ZROWS)])


_sc_agg = functools.partial(
    pl.kernel,
    out_type=jax.ShapeDtypeStruct((NC, NPAD, H), jnp.float32),
    mesh=_mesh,
    scratch_types=[
        pltpu.VMEM((CHUNK,), jnp.int32),
        pltpu.VMEM((CHUNK,), jnp.int32),
        pltpu.VMEM((CHUNK, H), jnp.float32),
        pltpu.VMEM((16, H), jnp.float32),
        pltpu.VMEM_SHARED((NPAD, H), jnp.float32),
        pltpu.SemaphoreType.DMA,
    ],
)(_sc_agg_body)


BLK = 1000
NBLK = N // BLK
EPS = 1e-5


def _ln(h, g, b):
    m = jnp.mean(h, axis=-1, keepdims=True)
    v = jnp.mean((h - m) ** 2, axis=-1, keepdims=True)
    return (h - m) * lax.rsqrt(v + EPS) * g + b


def _enc_body(x_ref, d0_ref, d1_ref, we_ref, be_ref, g_ref, bn_ref, w0_ref,
              h_ref, hp_ref, dis_ref):
    xb = x_ref[...]
    h = jnp.dot(xb, we_ref[...], preferred_element_type=jnp.float32)
    h = h + be_ref[...]
    h = jax.nn.relu(_ln(h, g_ref[...], bn_ref[...]))
    dis = lax.rsqrt(1.0 + d0_ref[...] + d1_ref[...])
    hp = jnp.dot(h, w0_ref[...], preferred_element_type=jnp.float32) * dis
    h_ref[...] = h
    hp_ref[...] = hp
    dis_ref[...] = dis


def _row_spec(w):
    return pl.BlockSpec((BLK, w), lambda b: (b, 0))


def _const_spec(shape):
    return pl.BlockSpec(shape, lambda b: tuple(0 for _ in shape))


def _tc_encoder(x, d0, d1, we, be, g, bn, w0):
    return pl.pallas_call(
        _enc_body,
        grid=(NBLK,),
        in_specs=[_row_spec(H), _row_spec(1), _row_spec(1),
                  _const_spec((H, H)), _const_spec((1, H)),
                  _const_spec((1, H)), _const_spec((1, H)),
                  _const_spec((H, H))],
        out_specs=[_row_spec(H), _row_spec(H), _row_spec(1)],
        out_shape=[jax.ShapeDtypeStruct((N, H), jnp.float32),
                   jax.ShapeDtypeStruct((N, H), jnp.float32),
                   jax.ShapeDtypeStruct((N, 1), jnp.float32)],
    )(x, d0, d1, we, be, g, bn, w0)


def _layer_body(h_ref, hp_ref, a0_ref, a1_ref, dis_ref, bc_ref, gc_ref,
                bn_ref, wn_ref, h2_ref, hp2_ref):
    dis = dis_ref[...]
    o = (a0_ref[...] + a1_ref[...] + hp_ref[...]) * dis + bc_ref[...]
    h2 = jax.nn.relu(_ln(o, gc_ref[...], bn_ref[...])) + h_ref[...]
    hp2 = jnp.dot(h2, wn_ref[...], preferred_element_type=jnp.float32) * dis
    h2_ref[...] = h2
    hp2_ref[...] = hp2


def _tc_layer(h, hp, a0, a1, dis, bc, gc, bn, wn):
    return pl.pallas_call(
        _layer_body,
        grid=(NBLK,),
        in_specs=[_row_spec(H), _row_spec(H), _row_spec(H), _row_spec(H),
                  _row_spec(1), _const_spec((1, H)), _const_spec((1, H)),
                  _const_spec((1, H)), _const_spec((H, H))],
        out_specs=[_row_spec(H), _row_spec(H)],
        out_shape=[jax.ShapeDtypeStruct((N, H), jnp.float32),
                   jax.ShapeDtypeStruct((N, H), jnp.float32)],
    )(h, hp, a0, a1, dis, bc, gc, bn, wn)


def _last_body(h_ref, hp_ref, a0_ref, a1_ref, dis_ref, bc_ref, gc_ref,
               bn_ref, h2_ref):
    o = (a0_ref[...] + a1_ref[...] + hp_ref[...]) * dis_ref[...] + bc_ref[...]
    h2_ref[...] = jax.nn.relu(_ln(o, gc_ref[...], bn_ref[...])) + h_ref[...]


def _tc_last_layer(h, hp, a0, a1, dis, bc, gc, bn):
    return pl.pallas_call(
        _last_body,
        grid=(NBLK,),
        in_specs=[_row_spec(H), _row_spec(H), _row_spec(H), _row_spec(H),
                  _row_spec(1), _const_spec((1, H)), _const_spec((1, H)),
                  _const_spec((1, H))],
        out_specs=[_row_spec(H)],
        out_shape=[jax.ShapeDtypeStruct((N, H), jnp.float32)],
    )(h, hp, a0, a1, dis, bc, gc, bn)


def _pool_body(h_ref, b_ref, gf_ref, wg1_ref, bg1_ref, wg2_ref, bg2_ref,
               wd1_ref, bd1_ref, gd_ref, bd_ref, wd2_ref, bd2_ref,
               wd3_ref, bd3_ref, out_ref, ssum, scnt, smax):
    b = pl.program_id(0)
    hb = h_ref[...]
    bb = b_ref[...]

    @pl.when(b == 0)
    def _():
        ssum[...] = jnp.zeros((G, H), jnp.float32)
        scnt[...] = jnp.zeros((G, 1), jnp.float32)
        smax[...] = jnp.full((G, H), -jnp.inf, jnp.float32)

    oh = (bb == lax.broadcasted_iota(jnp.int32, (BLK, G), 1))
    ohf = oh.astype(jnp.float32)
    dn = (((0,), (0,)), ((), ()))
    ssum[...] += lax.dot_general(ohf, hb, dn,
                                 preferred_element_type=jnp.float32)
    scnt[...] += lax.dot_general(ohf, jnp.ones((BLK, 1), jnp.float32), dn,
                                 preferred_element_type=jnp.float32)

    g0 = bb[0, 0]
    g1 = bb[BLK - 1, 0]
    for g in range(G):
        @pl.when((g0 <= g) & (g <= g1))
        def _():
            m = jnp.where(bb == g, hb, -jnp.inf)
            mg = jnp.max(m, axis=0, keepdims=True)
            smax[g:g + 1, :] = jnp.maximum(smax[g:g + 1, :], mg)

    @pl.when(b == NBLK - 1)
    def _():
        cnt = jnp.maximum(scnt[...], 1.0)
        mean = ssum[...] / cnt
        sm = smax[...]
        ss = ssum[...]
        ge = jnp.dot(
            jax.nn.relu(jnp.dot(gf_ref[...], wg1_ref[...],
                                preferred_element_type=jnp.float32)
                        + bg1_ref[...]),
            wg2_ref[...], preferred_element_type=jnp.float32) + bg2_ref[...]
        zw = (jnp.dot(mean, wd1_ref[0:H, :],
                      preferred_element_type=jnp.float32)
              + jnp.dot(sm, wd1_ref[H:2 * H, :],
                        preferred_element_type=jnp.float32)
              + jnp.dot(ss, wd1_ref[2 * H:3 * H, :],
                        preferred_element_type=jnp.float32)
              + jnp.dot(ge, wd1_ref[3 * H:4 * H, :],
                        preferred_element_type=jnp.float32)
              + bd1_ref[...])
        d1 = jax.nn.relu(_ln(zw, gd_ref[...], bd_ref[...]))
        d2 = jax.nn.relu(jnp.dot(d1, wd2_ref[...],
                                 preferred_element_type=jnp.float32)
                         + bd2_ref[...])
        lg = jnp.dot(d2, wd3_ref[...],
                     preferred_element_type=jnp.float32) + bd3_ref[...]
        mx = jnp.max(lg, axis=-1, keepdims=True)
        e = jnp.exp(lg - mx)
        out_ref[...] = e / jnp.sum(e, axis=-1, keepdims=True)


def _tc_pool_decode(h4, batch2d, gf, p):
    OUT = p['Wd3'].shape[1]
    return pl.pallas_call(
        _pool_body,
        grid=(NBLK,),
        in_specs=[_row_spec(H),
                  pl.BlockSpec((BLK, 1), lambda b: (b, 0)),
                  _const_spec((G, 4)),
                  _const_spec((4, H // 2)), _const_spec((1, H // 2)),
                  _const_spec((H // 2, H)), _const_spec((1, H)),
                  _const_spec((4 * H, 2 * H)), _const_spec((1, 2 * H)),
                  _const_spec((1, 2 * H)), _const_spec((1, 2 * H)),
                  _const_spec((2 * H, H)), _const_spec((1, H)),
                  _const_spec((H, OUT)), _const_spec((1, OUT))],
        out_specs=[pl.BlockSpec((G, OUT), lambda b: (0, 0))],
        out_shape=[jax.ShapeDtypeStruct((G, OUT), jnp.float32)],
        scratch_shapes=[pltpu.VMEM((G, H), jnp.float32),
                        pltpu.VMEM((G, 1), jnp.float32),
                        pltpu.VMEM((G, H), jnp.float32)],
    )(h4, batch2d,
      gf, p['Wg1'], p['bg1'].reshape(1, -1), p['Wg2'], p['bg2'].reshape(1, -1),
      p['Wd1'], p['bd1'].reshape(1, -1), p['gd'].reshape(1, -1),
      p['bd'].reshape(1, -1), p['Wd2'], p['bd2'].reshape(1, -1),
      p['Wd3'], p['bd3'].reshape(1, -1))[0]


def kernel(x, edge_index, batch, global_features, params):
    p = params
    src = edge_index[0]
    dst = edge_index[1]

    degp = _sc_degree(dst)
    d0 = degp[0].reshape(NPAD, 1)
    d1 = degp[1].reshape(NPAD, 1)

    h, hp, dis = _tc_encoder(
        x, d0, d1, p['W_enc'], p['b_enc'].reshape(1, -1),
        p['g_enc'].reshape(1, -1), p['be_enc'].reshape(1, -1), p['Wc'][0])

    for i in range(NL):
        agg = _sc_agg(hp, src, dst)
        bc = p['bc'][i].reshape(1, -1)
        gc = p['gc'][i].reshape(1, -1)
        bn = p['bnc'][i].reshape(1, -1)
        if i < NL - 1:
            h, hp = _tc_layer(h, hp, agg[0], agg[1], dis, bc, gc, bn,
                              p['Wc'][i + 1])
        else:
            h = _tc_last_layer(h, hp, agg[0], agg[1], dis, bc, gc, bn)[0]

    return _tc_pool_decode(h, batch.reshape(N, 1), global_features, p)


# speedup vs baseline: 22.3282x; 2.2469x over previous
import functools

import jax
import jax.numpy as jnp
from jax import lax
from jax.experimental import pallas as pl
from jax.experimental.pallas import tpu as pltpu
from jax.experimental.pallas import tpu_sc as plsc

N = 10000
E = 320000
H = 128
G = 64
NL = 4

NC = 2
NS = 16
NW = NC * NS
EW = E // NW
CHUNK = 80
NCHUNK = EW // CHUNK
NPAD = 10240
ZROWS = NPAD // NS

_mesh = plsc.VectorSubcoreMesh(core_axis_name="c", subcore_axis_name="s")


def _fill_1d(ref, n, value):
    for j in range(n // 16):
        ref[pl.ds(16 * j, 16)] = jnp.full((16,), value, jnp.float32)


def _sc_degree_body(dst_hbm, out_hbm, didx2, ones, zbuf, deg_acc, sem):
    c = lax.axis_index("c")
    s = lax.axis_index("s")
    wid = s * NC + c

    _fill_1d(ones, CHUNK, 1.0)
    _fill_1d(zbuf, ZROWS, 0.0)
    pltpu.sync_copy(dst_hbm.at[pl.ds(wid * EW, EW)], didx2)
    pltpu.sync_copy(zbuf, deg_acc.at[pl.ds(ZROWS * s, ZROWS)])
    plsc.subcore_barrier()

    FIRE = 5

    def body(k, _):
        for j in range(FIRE):
            i = FIRE * k + j
            pltpu.async_copy(
                ones, deg_acc.at[didx2.at[pl.ds(CHUNK * i, CHUNK)]], sem,
                add=True)
        for j in range(FIRE):
            pltpu.make_async_copy(
                ones, deg_acc.at[didx2.at[pl.ds(0, CHUNK)]], sem).wait()
        return ()

    lax.fori_loop(0, NCHUNK // FIRE, body, (), unroll=False)
    plsc.subcore_barrier()
    pltpu.sync_copy(deg_acc.at[pl.ds(ZROWS * s, ZROWS)],
                    out_hbm.at[c, pl.ds(ZROWS * s, ZROWS)])


_sc_degree = functools.partial(
    pl.kernel,
    out_type=jax.ShapeDtypeStruct((NC, NPAD), jnp.float32),
    mesh=_mesh,
    scratch_types=[
        pltpu.VMEM((EW,), jnp.int32),
        pltpu.VMEM((CHUNK,), jnp.float32),
        pltpu.VMEM((ZROWS,), jnp.float32),
        pltpu.VMEM_SHARED((NPAD,), jnp.float32),
        pltpu.SemaphoreType.DMA,
    ],
)(_sc_degree_body)


def _sc_agg_body(hp_hbm, src_hbm, dst_hbm, out_hbm,
                 sidx2, didx2, rows0, rows1, zb, acc, sem0, sem1):
    c = lax.axis_index("c")
    s = lax.axis_index("s")
    wid = s * NC + c

    for i in range(16):
        for j in range(8):
            zb[i, pl.ds(16 * j, 16)] = jnp.zeros((16,), jnp.float32)

    pltpu.sync_copy(src_hbm.at[pl.ds(wid * EW, EW)], sidx2)
    pltpu.sync_copy(dst_hbm.at[pl.ds(wid * EW, EW)], didx2)

    def zbody(k, _):
        pltpu.sync_copy(zb, acc.at[pl.ds(ZROWS * s + 16 * k, 16)])
        return ()

    lax.fori_loop(0, ZROWS // 16, zbody, (), unroll=False)
    plsc.subcore_barrier()

    def _fire(i, buf, sem):
        pltpu.async_copy(hp_hbm.at[sidx2.at[pl.ds(CHUNK * i, CHUNK)]],
                         buf, sem)

    def _wait(buf, sem):
        pltpu.make_async_copy(hp_hbm.at[sidx2.at[pl.ds(0, CHUNK)]],
                              buf, sem).wait()

    def _scat(i, buf):
        pltpu.sync_copy(buf, acc.at[didx2.at[pl.ds(CHUNK * i, CHUNK)]],
                        add=True)

    _fire(0, rows0, sem0)

    def body(k, _):
        a = 2 * k
        _fire(a + 1, rows1, sem1)
        _wait(rows0, sem0)
        _scat(a, rows0)
        _fire(a + 2, rows0, sem0)
        _wait(rows1, sem1)
        _scat(a + 1, rows1)
        return ()

    lax.fori_loop(0, (NCHUNK - 1) // 2, body, (), unroll=False)
    _wait(rows0, sem0)
    _scat(NCHUNK - 1, rows0)

    plsc.subcore_barrier()
    pltpu.sync_copy(acc.at[pl.ds(ZROWS * s, ZROWS)],
                    out_hbm.at[c, pl.ds(ZROWS * s, ZROWS)])


_sc_agg = functools.partial(
    pl.kernel,
    out_type=jax.ShapeDtypeStruct((NC, NPAD, H), jnp.float32),
    mesh=_mesh,
    scratch_types=[
        pltpu.VMEM((EW,), jnp.int32),
        pltpu.VMEM((EW,), jnp.int32),
        pltpu.VMEM((CHUNK, H), jnp.float32),
        pltpu.VMEM((CHUNK, H), jnp.float32),
        pltpu.VMEM((16, H), jnp.float32),
        pltpu.VMEM_SHARED((NPAD, H), jnp.float32),
        pltpu.SemaphoreType.DMA,
        pltpu.SemaphoreType.DMA,
    ],
)(_sc_agg_body)


BLK = 1000
NBLK = N // BLK
EPS = 1e-5


def _ln(h, g, b):
    m = jnp.mean(h, axis=-1, keepdims=True)
    v = jnp.mean((h - m) ** 2, axis=-1, keepdims=True)
    return (h - m) * lax.rsqrt(v + EPS) * g + b


def _enc_body(x_ref, d0_ref, d1_ref, we_ref, be_ref, g_ref, bn_ref, w0_ref,
              h_ref, hp_ref, dis_ref):
    xb = x_ref[...]
    h = jnp.dot(xb, we_ref[...], preferred_element_type=jnp.float32)
    h = h + be_ref[...]
    h = jax.nn.relu(_ln(h, g_ref[...], bn_ref[...]))
    dis = lax.rsqrt(1.0 + d0_ref[...] + d1_ref[...])
    hp = jnp.dot(h, w0_ref[...], preferred_element_type=jnp.float32) * dis
    h_ref[...] = h
    hp_ref[...] = hp
    dis_ref[...] = dis


def _row_spec(w):
    return pl.BlockSpec((BLK, w), lambda b: (b, 0))


def _const_spec(shape):
    return pl.BlockSpec(shape, lambda b: tuple(0 for _ in shape))


def _tc_encoder(x, d0, d1, we, be, g, bn, w0):
    return pl.pallas_call(
        _enc_body,
        grid=(NBLK,),
        in_specs=[_row_spec(H), _row_spec(1), _row_spec(1),
                  _const_spec((H, H)), _const_spec((1, H)),
                  _const_spec((1, H)), _const_spec((1, H)),
                  _const_spec((H, H))],
        out_specs=[_row_spec(H), _row_spec(H), _row_spec(1)],
        out_shape=[jax.ShapeDtypeStruct((N, H), jnp.float32),
                   jax.ShapeDtypeStruct((N, H), jnp.float32),
                   jax.ShapeDtypeStruct((N, 1), jnp.float32)],
    )(x, d0, d1, we, be, g, bn, w0)


def _layer_body(h_ref, hp_ref, a0_ref, a1_ref, dis_ref, bc_ref, gc_ref,
                bn_ref, wn_ref, h2_ref, hp2_ref):
    dis = dis_ref[...]
    o = (a0_ref[...] + a1_ref[...] + hp_ref[...]) * dis + bc_ref[...]
    h2 = jax.nn.relu(_ln(o, gc_ref[...], bn_ref[...])) + h_ref[...]
    hp2 = jnp.dot(h2, wn_ref[...], preferred_element_type=jnp.float32) * dis
    h2_ref[...] = h2
    hp2_ref[...] = hp2


def _tc_layer(h, hp, a0, a1, dis, bc, gc, bn, wn):
    return pl.pallas_call(
        _layer_body,
        grid=(NBLK,),
        in_specs=[_row_spec(H), _row_spec(H), _row_spec(H), _row_spec(H),
                  _row_spec(1), _const_spec((1, H)), _const_spec((1, H)),
                  _const_spec((1, H)), _const_spec((H, H))],
        out_specs=[_row_spec(H), _row_spec(H)],
        out_shape=[jax.ShapeDtypeStruct((N, H), jnp.float32),
                   jax.ShapeDtypeStruct((N, H), jnp.float32)],
    )(h, hp, a0, a1, dis, bc, gc, bn, wn)


def _last_body(h_ref, hp_ref, a0_ref, a1_ref, dis_ref, bc_ref, gc_ref,
               bn_ref, h2_ref):
    o = (a0_ref[...] + a1_ref[...] + hp_ref[...]) * dis_ref[...] + bc_ref[...]
    h2_ref[...] = jax.nn.relu(_ln(o, gc_ref[...], bn_ref[...])) + h_ref[...]


def _tc_last_layer(h, hp, a0, a1, dis, bc, gc, bn):
    return pl.pallas_call(
        _last_body,
        grid=(NBLK,),
        in_specs=[_row_spec(H), _row_spec(H), _row_spec(H), _row_spec(H),
                  _row_spec(1), _const_spec((1, H)), _const_spec((1, H)),
                  _const_spec((1, H))],
        out_specs=[_row_spec(H)],
        out_shape=[jax.ShapeDtypeStruct((N, H), jnp.float32)],
    )(h, hp, a0, a1, dis, bc, gc, bn)


def _pool_body(h_ref, b_ref, gf_ref, wg1_ref, bg1_ref, wg2_ref, bg2_ref,
               wd1_ref, bd1_ref, gd_ref, bd_ref, wd2_ref, bd2_ref,
               wd3_ref, bd3_ref, out_ref, ssum, scnt, smax):
    b = pl.program_id(0)
    hb = h_ref[...]
    bb = b_ref[...]

    @pl.when(b == 0)
    def _():
        ssum[...] = jnp.zeros((G, H), jnp.float32)
        scnt[...] = jnp.zeros((G, 1), jnp.float32)
        smax[...] = jnp.full((G, H), -jnp.inf, jnp.float32)

    oh = (bb == lax.broadcasted_iota(jnp.int32, (BLK, G), 1))
    ohf = oh.astype(jnp.float32)
    dn = (((0,), (0,)), ((), ()))
    ssum[...] += lax.dot_general(ohf, hb, dn,
                                 preferred_element_type=jnp.float32)
    scnt[...] += lax.dot_general(ohf, jnp.ones((BLK, 1), jnp.float32), dn,
                                 preferred_element_type=jnp.float32)

    g0 = bb[0, 0]
    g1 = bb[BLK - 1, 0]
    for g in range(G):
        @pl.when((g0 <= g) & (g <= g1))
        def _():
            m = jnp.where(bb == g, hb, -jnp.inf)
            mg = jnp.max(m, axis=0, keepdims=True)
            smax[g:g + 1, :] = jnp.maximum(smax[g:g + 1, :], mg)

    @pl.when(b == NBLK - 1)
    def _():
        cnt = jnp.maximum(scnt[...], 1.0)
        mean = ssum[...] / cnt
        sm = smax[...]
        ss = ssum[...]
        ge = jnp.dot(
            jax.nn.relu(jnp.dot(gf_ref[...], wg1_ref[...],
                                preferred_element_type=jnp.float32)
                        + bg1_ref[...]),
            wg2_ref[...], preferred_element_type=jnp.float32) + bg2_ref[...]
        zw = (jnp.dot(mean, wd1_ref[0:H, :],
                      preferred_element_type=jnp.float32)
              + jnp.dot(sm, wd1_ref[H:2 * H, :],
                        preferred_element_type=jnp.float32)
              + jnp.dot(ss, wd1_ref[2 * H:3 * H, :],
                        preferred_element_type=jnp.float32)
              + jnp.dot(ge, wd1_ref[3 * H:4 * H, :],
                        preferred_element_type=jnp.float32)
              + bd1_ref[...])
        d1 = jax.nn.relu(_ln(zw, gd_ref[...], bd_ref[...]))
        d2 = jax.nn.relu(jnp.dot(d1, wd2_ref[...],
                                 preferred_element_type=jnp.float32)
                         + bd2_ref[...])
        lg = jnp.dot(d2, wd3_ref[...],
                     preferred_element_type=jnp.float32) + bd3_ref[...]
        mx = jnp.max(lg, axis=-1, keepdims=True)
        e = jnp.exp(lg - mx)
        out_ref[...] = e / jnp.sum(e, axis=-1, keepdims=True)


def _tc_pool_decode(h4, batch2d, gf, p):
    OUT = p['Wd3'].shape[1]
    return pl.pallas_call(
        _pool_body,
        grid=(NBLK,),
        in_specs=[_row_spec(H),
                  pl.BlockSpec((BLK, 1), lambda b: (b, 0)),
                  _const_spec((G, 4)),
                  _const_spec((4, H // 2)), _const_spec((1, H // 2)),
                  _const_spec((H // 2, H)), _const_spec((1, H)),
                  _const_spec((4 * H, 2 * H)), _const_spec((1, 2 * H)),
                  _const_spec((1, 2 * H)), _const_spec((1, 2 * H)),
                  _const_spec((2 * H, H)), _const_spec((1, H)),
                  _const_spec((H, OUT)), _const_spec((1, OUT))],
        out_specs=[pl.BlockSpec((G, OUT), lambda b: (0, 0))],
        out_shape=[jax.ShapeDtypeStruct((G, OUT), jnp.float32)],
        scratch_shapes=[pltpu.VMEM((G, H), jnp.float32),
                        pltpu.VMEM((G, 1), jnp.float32),
                        pltpu.VMEM((G, H), jnp.float32)],
    )(h4, batch2d,
      gf, p['Wg1'], p['bg1'].reshape(1, -1), p['Wg2'], p['bg2'].reshape(1, -1),
      p['Wd1'], p['bd1'].reshape(1, -1), p['gd'].reshape(1, -1),
      p['bd'].reshape(1, -1), p['Wd2'], p['bd2'].reshape(1, -1),
      p['Wd3'], p['bd3'].reshape(1, -1))[0]


def kernel(x, edge_index, batch, global_features, params):
    p = params
    src = edge_index[0]
    dst = edge_index[1]

    degp = _sc_degree(dst)
    d0 = degp[0].reshape(NPAD, 1)
    d1 = degp[1].reshape(NPAD, 1)

    h, hp, dis = _tc_encoder(
        x, d0, d1, p['W_enc'], p['b_enc'].reshape(1, -1),
        p['g_enc'].reshape(1, -1), p['be_enc'].reshape(1, -1), p['Wc'][0])

    for i in range(NL):
        agg = _sc_agg(hp, src, dst)
        bc = p['bc'][i].reshape(1, -1)
        gc = p['gc'][i].reshape(1, -1)
        bn = p['bnc'][i].reshape(1, -1)
        if i < NL - 1:
            h, hp = _tc_layer(h, hp, agg[0], agg[1], dis, bc, gc, bn,
                              p['Wc'][i + 1])
        else:
            h = _tc_last_layer(h, hp, agg[0], agg[1], dis, bc, gc, bn)[0]

    return _tc_pool_decode(h, batch.reshape(N, 1), global_features, p)


# speedup vs baseline: 24.0884x; 1.0788x over previous
import functools

import jax
import jax.numpy as jnp
from jax import lax
from jax.experimental import pallas as pl
from jax.experimental.pallas import tpu as pltpu
from jax.experimental.pallas import tpu_sc as plsc

N = 10000
E = 320000
H = 128
G = 64
NL = 4

NC = 2
NS = 16
NW = NC * NS
CHUNK = 128
NCHUNK = 80
EW = NCHUNK * CHUNK
EPADN = NW * EW - E
NPAD = 10240
ZROWS = NPAD // NS

_mesh = plsc.VectorSubcoreMesh(core_axis_name="c", subcore_axis_name="s")


def _fill_1d(ref, n, value):
    for j in range(n // 16):
        ref[pl.ds(16 * j, 16)] = jnp.full((16,), value, jnp.float32)


def _sc_degree_body(dst_hbm, out_hbm, didx2, ones, zbuf, deg_acc, sem):
    c = lax.axis_index("c")
    s = lax.axis_index("s")
    wid = s * NC + c

    _fill_1d(ones, CHUNK, 1.0)
    _fill_1d(zbuf, ZROWS, 0.0)
    pltpu.sync_copy(dst_hbm.at[pl.ds(wid * EW, EW)], didx2)
    pltpu.sync_copy(zbuf, deg_acc.at[pl.ds(ZROWS * s, ZROWS)])
    plsc.subcore_barrier()

    FIRE = 8

    def body(k, _):
        for j in range(FIRE):
            i = FIRE * k + j
            pltpu.async_copy(
                ones, deg_acc.at[didx2.at[pl.ds(CHUNK * i, CHUNK)]], sem,
                add=True)
        for j in range(FIRE):
            pltpu.make_async_copy(
                ones, deg_acc.at[didx2.at[pl.ds(0, CHUNK)]], sem).wait()
        return ()

    lax.fori_loop(0, NCHUNK // FIRE, body, (), unroll=False)
    plsc.subcore_barrier()
    pltpu.sync_copy(deg_acc.at[pl.ds(ZROWS * s, ZROWS)],
                    out_hbm.at[c, pl.ds(ZROWS * s, ZROWS)])


_sc_degree = functools.partial(
    pl.kernel,
    out_type=jax.ShapeDtypeStruct((NC, NPAD), jnp.float32),
    mesh=_mesh,
    scratch_types=[
        pltpu.VMEM((EW,), jnp.int32),
        pltpu.VMEM((CHUNK,), jnp.float32),
        pltpu.VMEM((ZROWS,), jnp.float32),
        pltpu.VMEM_SHARED((NPAD,), jnp.float32),
        pltpu.SemaphoreType.DMA,
    ],
)(_sc_degree_body)


def _sc_agg_body(hp_hbm, src_hbm, dst_hbm, out_hbm,
                 didx2, sbufs, rows, zb, acc, isems, gsems):
    c = lax.axis_index("c")
    s = lax.axis_index("s")
    wid = s * NC + c

    for i in range(16):
        for j in range(8):
            zb[i, pl.ds(16 * j, 16)] = jnp.zeros((16,), jnp.float32)

    pltpu.sync_copy(dst_hbm.at[pl.ds(wid * EW, EW)], didx2)

    for k in range(ZROWS // 16):
        pltpu.async_copy(zb, acc.at[pl.ds(ZROWS * s + 16 * k, 16)],
                         gsems[0])
    for k in range(ZROWS // 16):
        pltpu.make_async_copy(zb, acc.at[pl.ds(0, 16)], gsems[0]).wait()
    plsc.subcore_barrier()

    def _ifire(i, q):
        pltpu.async_copy(src_hbm.at[pl.ds(wid * EW + CHUNK * i, CHUNK)],
                         sbufs.at[q], isems[q])

    def _fire(i, q):
        pltpu.make_async_copy(src_hbm.at[pl.ds(0, CHUNK)],
                              sbufs.at[q], isems[q]).wait()
        pltpu.async_copy(hp_hbm.at[sbufs.at[q]], rows.at[q], gsems[q])

    def _gwait(q):
        pltpu.make_async_copy(hp_hbm.at[sbufs.at[0]],
                              rows.at[q], gsems[q]).wait()

    def _scat(i, q):
        pltpu.sync_copy(rows.at[q],
                        acc.at[didx2.at[pl.ds(CHUNK * i, CHUNK)]],
                        add=True)

    _ifire(0, 0)
    _ifire(1, 1)
    _fire(0, 0)

    def body(k, _):
        a = 2 * k
        _fire(a + 1, 1)
        _gwait(0)
        _ifire(a + 2, 0)
        _scat(a, 0)
        _fire(a + 2, 0)
        _gwait(1)
        _ifire(a + 3, 1)
        _scat(a + 1, 1)
        return ()

    lax.fori_loop(0, NCHUNK // 2 - 1, body, (), unroll=False)
    _fire(NCHUNK - 1, 1)
    _gwait(0)
    _scat(NCHUNK - 2, 0)
    _gwait(1)
    _scat(NCHUNK - 1, 1)

    plsc.subcore_barrier()
    pltpu.sync_copy(acc.at[pl.ds(ZROWS * s, ZROWS)],
                    out_hbm.at[c, pl.ds(ZROWS * s, ZROWS)])


_sc_agg = functools.partial(
    pl.kernel,
    out_type=jax.ShapeDtypeStruct((NC, NPAD, H), jnp.float32),
    mesh=_mesh,
    scratch_types=[
        pltpu.VMEM((EW,), jnp.int32),
        pltpu.VMEM((2, CHUNK), jnp.int32),
        pltpu.VMEM((2, CHUNK, H), jnp.float32),
        pltpu.VMEM((16, H), jnp.float32),
        pltpu.VMEM_SHARED((NPAD, H), jnp.float32),
        [pltpu.SemaphoreType.DMA] * 2,
        [pltpu.SemaphoreType.DMA] * 2,
    ],
)(_sc_agg_body)


BLK = 1000
NBLK = N // BLK
EPS = 1e-5


def _ln(h, g, b):
    m = jnp.mean(h, axis=-1, keepdims=True)
    v = jnp.mean((h - m) ** 2, axis=-1, keepdims=True)
    return (h - m) * lax.rsqrt(v + EPS) * g + b


def _enc_body(x_ref, d0_ref, d1_ref, we_ref, be_ref, g_ref, bn_ref, w0_ref,
              h_ref, hp_ref, dis_ref):
    xb = x_ref[...]
    h = jnp.dot(xb, we_ref[...], preferred_element_type=jnp.float32)
    h = h + be_ref[...]
    h = jax.nn.relu(_ln(h, g_ref[...], bn_ref[...]))
    dis = lax.rsqrt(1.0 + d0_ref[...] + d1_ref[...])
    hp = jnp.dot(h, w0_ref[...], preferred_element_type=jnp.float32) * dis
    h_ref[...] = h
    hp_ref[...] = hp
    dis_ref[...] = dis


def _row_spec(w):
    return pl.BlockSpec((BLK, w), lambda b: (b, 0))


def _const_spec(shape):
    return pl.BlockSpec(shape, lambda b: tuple(0 for _ in shape))


def _tc_encoder(x, d0, d1, we, be, g, bn, w0):
    return pl.pallas_call(
        _enc_body,
        grid=(NBLK,),
        in_specs=[_row_spec(H), _row_spec(1), _row_spec(1),
                  _const_spec((H, H)), _const_spec((1, H)),
                  _const_spec((1, H)), _const_spec((1, H)),
                  _const_spec((H, H))],
        out_specs=[_row_spec(H), _row_spec(H), _row_spec(1)],
        out_shape=[jax.ShapeDtypeStruct((N, H), jnp.float32),
                   jax.ShapeDtypeStruct((N, H), jnp.float32),
                   jax.ShapeDtypeStruct((N, 1), jnp.float32)],
    )(x, d0, d1, we, be, g, bn, w0)


def _layer_body(h_ref, hp_ref, a0_ref, a1_ref, dis_ref, bc_ref, gc_ref,
                bn_ref, wn_ref, h2_ref, hp2_ref):
    dis = dis_ref[...]
    o = (a0_ref[...] + a1_ref[...] + hp_ref[...]) * dis + bc_ref[...]
    h2 = jax.nn.relu(_ln(o, gc_ref[...], bn_ref[...])) + h_ref[...]
    hp2 = jnp.dot(h2, wn_ref[...], preferred_element_type=jnp.float32) * dis
    h2_ref[...] = h2
    hp2_ref[...] = hp2


def _tc_layer(h, hp, a0, a1, dis, bc, gc, bn, wn):
    return pl.pallas_call(
        _layer_body,
        grid=(NBLK,),
        in_specs=[_row_spec(H), _row_spec(H), _row_spec(H), _row_spec(H),
                  _row_spec(1), _const_spec((1, H)), _const_spec((1, H)),
                  _const_spec((1, H)), _const_spec((H, H))],
        out_specs=[_row_spec(H), _row_spec(H)],
        out_shape=[jax.ShapeDtypeStruct((N, H), jnp.float32),
                   jax.ShapeDtypeStruct((N, H), jnp.float32)],
    )(h, hp, a0, a1, dis, bc, gc, bn, wn)


def _last_body(h_ref, hp_ref, a0_ref, a1_ref, dis_ref, bc_ref, gc_ref,
               bn_ref, h2_ref):
    o = (a0_ref[...] + a1_ref[...] + hp_ref[...]) * dis_ref[...] + bc_ref[...]
    h2_ref[...] = jax.nn.relu(_ln(o, gc_ref[...], bn_ref[...])) + h_ref[...]


def _tc_last_layer(h, hp, a0, a1, dis, bc, gc, bn):
    return pl.pallas_call(
        _last_body,
        grid=(NBLK,),
        in_specs=[_row_spec(H), _row_spec(H), _row_spec(H), _row_spec(H),
                  _row_spec(1), _const_spec((1, H)), _const_spec((1, H)),
                  _const_spec((1, H))],
        out_specs=[_row_spec(H)],
        out_shape=[jax.ShapeDtypeStruct((N, H), jnp.float32)],
    )(h, hp, a0, a1, dis, bc, gc, bn)


def _pool_body(h_ref, b_ref, gf_ref, wg1_ref, bg1_ref, wg2_ref, bg2_ref,
               wd1_ref, bd1_ref, gd_ref, bd_ref, wd2_ref, bd2_ref,
               wd3_ref, bd3_ref, out_ref, ssum, scnt, smax):
    b = pl.program_id(0)
    hb = h_ref[...]
    bb = b_ref[...]

    @pl.when(b == 0)
    def _():
        ssum[...] = jnp.zeros((G, H), jnp.float32)
        scnt[...] = jnp.zeros((G, 1), jnp.float32)
        smax[...] = jnp.full((G, H), -jnp.inf, jnp.float32)

    oh = (bb == lax.broadcasted_iota(jnp.int32, (BLK, G), 1))
    ohf = oh.astype(jnp.float32)
    dn = (((0,), (0,)), ((), ()))
    ssum[...] += lax.dot_general(ohf, hb, dn,
                                 preferred_element_type=jnp.float32)
    scnt[...] += lax.dot_general(ohf, jnp.ones((BLK, 1), jnp.float32), dn,
                                 preferred_element_type=jnp.float32)

    g0 = bb[0, 0]
    g1 = bb[BLK - 1, 0]
    for g in range(G):
        @pl.when((g0 <= g) & (g <= g1))
        def _():
            m = jnp.where(bb == g, hb, -jnp.inf)
            mg = jnp.max(m, axis=0, keepdims=True)
            smax[g:g + 1, :] = jnp.maximum(smax[g:g + 1, :], mg)

    @pl.when(b == NBLK - 1)
    def _():
        cnt = jnp.maximum(scnt[...], 1.0)
        mean = ssum[...] / cnt
        sm = smax[...]
        ss = ssum[...]
        ge = jnp.dot(
            jax.nn.relu(jnp.dot(gf_ref[...], wg1_ref[...],
                                preferred_element_type=jnp.float32)
                        + bg1_ref[...]),
            wg2_ref[...], preferred_element_type=jnp.float32) + bg2_ref[...]
        zw = (jnp.dot(mean, wd1_ref[0:H, :],
                      preferred_element_type=jnp.float32)
              + jnp.dot(sm, wd1_ref[H:2 * H, :],
                        preferred_element_type=jnp.float32)
              + jnp.dot(ss, wd1_ref[2 * H:3 * H, :],
                        preferred_element_type=jnp.float32)
              + jnp.dot(ge, wd1_ref[3 * H:4 * H, :],
                        preferred_element_type=jnp.float32)
              + bd1_ref[...])
        d1 = jax.nn.relu(_ln(zw, gd_ref[...], bd_ref[...]))
        d2 = jax.nn.relu(jnp.dot(d1, wd2_ref[...],
                                 preferred_element_type=jnp.float32)
                         + bd2_ref[...])
        lg = jnp.dot(d2, wd3_ref[...],
                     preferred_element_type=jnp.float32) + bd3_ref[...]
        mx = jnp.max(lg, axis=-1, keepdims=True)
        e = jnp.exp(lg - mx)
        out_ref[...] = e / jnp.sum(e, axis=-1, keepdims=True)


def _tc_pool_decode(h4, batch2d, gf, p):
    OUT = p['Wd3'].shape[1]
    return pl.pallas_call(
        _pool_body,
        grid=(NBLK,),
        in_specs=[_row_spec(H),
                  pl.BlockSpec((BLK, 1), lambda b: (b, 0)),
                  _const_spec((G, 4)),
                  _const_spec((4, H // 2)), _const_spec((1, H // 2)),
                  _const_spec((H // 2, H)), _const_spec((1, H)),
                  _const_spec((4 * H, 2 * H)), _const_spec((1, 2 * H)),
                  _const_spec((1, 2 * H)), _const_spec((1, 2 * H)),
                  _const_spec((2 * H, H)), _const_spec((1, H)),
                  _const_spec((H, OUT)), _const_spec((1, OUT))],
        out_specs=[pl.BlockSpec((G, OUT), lambda b: (0, 0))],
        out_shape=[jax.ShapeDtypeStruct((G, OUT), jnp.float32)],
        scratch_shapes=[pltpu.VMEM((G, H), jnp.float32),
                        pltpu.VMEM((G, 1), jnp.float32),
                        pltpu.VMEM((G, H), jnp.float32)],
    )(h4, batch2d,
      gf, p['Wg1'], p['bg1'].reshape(1, -1), p['Wg2'], p['bg2'].reshape(1, -1),
      p['Wd1'], p['bd1'].reshape(1, -1), p['gd'].reshape(1, -1),
      p['bd'].reshape(1, -1), p['Wd2'], p['bd2'].reshape(1, -1),
      p['Wd3'], p['bd3'].reshape(1, -1))[0]


def kernel(x, edge_index, batch, global_features, params):
    p = params
    pad = jnp.arange(EPADN, dtype=jnp.int32)
    src = jnp.concatenate([edge_index[0], pad % N])
    dst = jnp.concatenate([edge_index[1], N + pad % (NPAD - N)])

    degp = _sc_degree(dst)
    d0 = degp[0].reshape(NPAD, 1)
    d1 = degp[1].reshape(NPAD, 1)

    h, hp, dis = _tc_encoder(
        x, d0, d1, p['W_enc'], p['b_enc'].reshape(1, -1),
        p['g_enc'].reshape(1, -1), p['be_enc'].reshape(1, -1), p['Wc'][0])

    for i in range(NL):
        agg = _sc_agg(hp, src, dst)
        bc = p['bc'][i].reshape(1, -1)
        gc = p['gc'][i].reshape(1, -1)
        bn = p['bnc'][i].reshape(1, -1)
        if i < NL - 1:
            h, hp = _tc_layer(h, hp, agg[0], agg[1], dis, bc, gc, bn,
                              p['Wc'][i + 1])
        else:
            h = _tc_last_layer(h, hp, agg[0], agg[1], dis, bc, gc, bn)[0]

    return _tc_pool_decode(h, batch.reshape(N, 1), global_features, p)


# speedup vs baseline: 24.4725x; 1.0159x over previous
import functools

import jax
import jax.numpy as jnp
from jax import lax
from jax.experimental import pallas as pl
from jax.experimental.pallas import tpu as pltpu
from jax.experimental.pallas import tpu_sc as plsc

N = 10000
E = 320000
H = 128
G = 64
NL = 4

NC = 2
NS = 16
NW = NC * NS
CHUNK = 128
NCHUNK = 80
EW = NCHUNK * CHUNK
EPADN = NW * EW - E
NPAD = 10240
ZROWS = NPAD // NS

_mesh = plsc.VectorSubcoreMesh(core_axis_name="c", subcore_axis_name="s")


def _fill_1d(ref, n, value):
    for j in range(n // 16):
        ref[pl.ds(16 * j, 16)] = jnp.full((16,), value, jnp.float32)


def _sc_degree_body(dst_hbm, out_hbm, didx2, ones, zbuf, deg_acc, sem):
    c = lax.axis_index("c")
    s = lax.axis_index("s")
    wid = s * NC + c

    _fill_1d(ones, CHUNK, 1.0)
    _fill_1d(zbuf, ZROWS, 0.0)
    pltpu.sync_copy(dst_hbm.at[pl.ds(wid * EW, EW)], didx2)
    pltpu.sync_copy(zbuf, deg_acc.at[pl.ds(ZROWS * s, ZROWS)])
    plsc.subcore_barrier()

    FIRE = 8

    def body(k, _):
        for j in range(FIRE):
            i = FIRE * k + j
            pltpu.async_copy(
                ones, deg_acc.at[didx2.at[pl.ds(CHUNK * i, CHUNK)]], sem,
                add=True)
        for j in range(FIRE):
            pltpu.make_async_copy(
                ones, deg_acc.at[didx2.at[pl.ds(0, CHUNK)]], sem).wait()
        return ()

    lax.fori_loop(0, NCHUNK // FIRE, body, (), unroll=False)
    plsc.subcore_barrier()
    pltpu.sync_copy(deg_acc.at[pl.ds(ZROWS * s, ZROWS)],
                    out_hbm.at[c, pl.ds(ZROWS * s, ZROWS)])


_sc_degree = functools.partial(
    pl.kernel,
    out_type=jax.ShapeDtypeStruct((NC, NPAD), jnp.float32),
    mesh=_mesh,
    scratch_types=[
        pltpu.VMEM((EW,), jnp.int32),
        pltpu.VMEM((CHUNK,), jnp.float32),
        pltpu.VMEM((ZROWS,), jnp.float32),
        pltpu.VMEM_SHARED((NPAD,), jnp.float32),
        pltpu.SemaphoreType.DMA,
    ],
)(_sc_degree_body)


def _sc_agg_body(hp_hbm, src_hbm, dst_hbm, out_hbm,
                 didx2, sbufs, rows, zb, acc, isems, gsems):
    c = lax.axis_index("c")
    s = lax.axis_index("s")
    wid = s * NC + c

    for i in range(16):
        for j in range(8):
            zb[i, pl.ds(16 * j, 16)] = jnp.zeros((16,), jnp.float32)

    pltpu.sync_copy(dst_hbm.at[pl.ds(wid * EW, EW)], didx2)

    for k in range(ZROWS // 16):
        pltpu.async_copy(zb, acc.at[pl.ds(ZROWS * s + 16 * k, 16)],
                         gsems[0])
    for k in range(ZROWS // 16):
        pltpu.make_async_copy(zb, acc.at[pl.ds(0, 16)], gsems[0]).wait()
    plsc.subcore_barrier()

    def _ifire(i, q):
        pltpu.async_copy(src_hbm.at[pl.ds(wid * EW + CHUNK * i, CHUNK)],
                         sbufs.at[q], isems[q])

    def _fire(i, q):
        pltpu.make_async_copy(src_hbm.at[pl.ds(0, CHUNK)],
                              sbufs.at[q], isems[q]).wait()
        pltpu.async_copy(hp_hbm.at[sbufs.at[q]], rows.at[q], gsems[q])

    def _gwait(q):
        pltpu.make_async_copy(hp_hbm.at[sbufs.at[0]],
                              rows.at[q], gsems[q]).wait()

    def _scat(i, q):
        pltpu.sync_copy(rows.at[q],
                        acc.at[didx2.at[pl.ds(CHUNK * i, CHUNK)]],
                        add=True)

    _ifire(0, 0)
    _ifire(1, 1)
    _fire(0, 0)

    def body(k, _):
        a = 2 * k
        _fire(a + 1, 1)
        _gwait(0)
        _ifire(a + 2, 0)
        _scat(a, 0)
        _fire(a + 2, 0)
        _gwait(1)
        _ifire(a + 3, 1)
        _scat(a + 1, 1)
        return ()

    lax.fori_loop(0, NCHUNK // 2 - 1, body, (), unroll=False)
    _fire(NCHUNK - 1, 1)
    _gwait(0)
    _scat(NCHUNK - 2, 0)
    _gwait(1)
    _scat(NCHUNK - 1, 1)

    plsc.subcore_barrier()
    pltpu.sync_copy(acc.at[pl.ds(ZROWS * s, ZROWS)],
                    out_hbm.at[c, pl.ds(ZROWS * s, ZROWS)])


_sc_agg = functools.partial(
    pl.kernel,
    out_type=jax.ShapeDtypeStruct((NC, NPAD, H), jnp.float32),
    mesh=_mesh,
    scratch_types=[
        pltpu.VMEM((EW,), jnp.int32),
        pltpu.VMEM((2, CHUNK), jnp.int32),
        pltpu.VMEM((2, CHUNK, H), jnp.float32),
        pltpu.VMEM((16, H), jnp.float32),
        pltpu.VMEM_SHARED((NPAD, H), jnp.float32),
        [pltpu.SemaphoreType.DMA] * 2,
        [pltpu.SemaphoreType.DMA] * 2,
    ],
)(_sc_agg_body)


BLK = 1000
NBLK = N // BLK
EPS = 1e-5


def _ln(h, g, b):
    m = jnp.mean(h, axis=-1, keepdims=True)
    v = jnp.mean((h - m) ** 2, axis=-1, keepdims=True)
    return (h - m) * lax.rsqrt(v + EPS) * g + b


def _enc_body(x_ref, d0_ref, d1_ref, we_ref, be_ref, g_ref, bn_ref, w0_ref,
              h_ref, hp_ref, dis_ref):
    xb = x_ref[...]
    h = jnp.dot(xb, we_ref[...], preferred_element_type=jnp.float32)
    h = h + be_ref[...]
    h = jax.nn.relu(_ln(h, g_ref[...], bn_ref[...]))
    dis = lax.rsqrt(1.0 + d0_ref[...] + d1_ref[...])
    hp = jnp.dot(h, w0_ref[...], preferred_element_type=jnp.float32) * dis
    h_ref[...] = h
    hp_ref[...] = hp
    dis_ref[...] = dis


def _row_spec(w):
    return pl.BlockSpec((BLK, w), lambda b: (b, 0))


def _const_spec(shape):
    return pl.BlockSpec(shape, lambda b: tuple(0 for _ in shape))


def _tc_encoder(x, d0, d1, we, be, g, bn, w0):
    return pl.pallas_call(
        _enc_body,
        grid=(NBLK,),
        in_specs=[_row_spec(H), _row_spec(1), _row_spec(1),
                  _const_spec((H, H)), _const_spec((1, H)),
                  _const_spec((1, H)), _const_spec((1, H)),
                  _const_spec((H, H))],
        out_specs=[_row_spec(H), _row_spec(H), _row_spec(1)],
        out_shape=[jax.ShapeDtypeStruct((N, H), jnp.float32),
                   jax.ShapeDtypeStruct((N, H), jnp.float32),
                   jax.ShapeDtypeStruct((N, 1), jnp.float32)],
    )(x, d0, d1, we, be, g, bn, w0)


def _layer_body(h_ref, hp_ref, a0_ref, a1_ref, dis_ref, bc_ref, gc_ref,
                bn_ref, wn_ref, h2_ref, hp2_ref):
    dis = dis_ref[...]
    o = (a0_ref[...] + a1_ref[...] + hp_ref[...]) * dis + bc_ref[...]
    h2 = jax.nn.relu(_ln(o, gc_ref[...], bn_ref[...])) + h_ref[...]
    hp2 = jnp.dot(h2, wn_ref[...], preferred_element_type=jnp.float32) * dis
    h2_ref[...] = h2
    hp2_ref[...] = hp2


def _tc_layer(h, hp, a0, a1, dis, bc, gc, bn, wn):
    return pl.pallas_call(
        _layer_body,
        grid=(NBLK,),
        in_specs=[_row_spec(H), _row_spec(H), _row_spec(H), _row_spec(H),
                  _row_spec(1), _const_spec((1, H)), _const_spec((1, H)),
                  _const_spec((1, H)), _const_spec((H, H))],
        out_specs=[_row_spec(H), _row_spec(H)],
        out_shape=[jax.ShapeDtypeStruct((N, H), jnp.float32),
                   jax.ShapeDtypeStruct((N, H), jnp.float32)],
    )(h, hp, a0, a1, dis, bc, gc, bn, wn)


def _pool_body(h_ref, hp_ref, a0_ref, a1_ref, dis_ref, bc_ref, gc_ref,
               bn_ref, b_ref, gf_ref, wg1_ref, bg1_ref, wg2_ref, bg2_ref,
               wd1_ref, bd1_ref, gd_ref, bd_ref, wd2_ref, bd2_ref,
               wd3_ref, bd3_ref, out_ref, ssum, scnt, smax):
    b = pl.program_id(0)
    o = (a0_ref[...] + a1_ref[...] + hp_ref[...]) * dis_ref[...] + bc_ref[...]
    hb = jax.nn.relu(_ln(o, gc_ref[...], bn_ref[...])) + h_ref[...]
    bb = b_ref[...]

    @pl.when(b == 0)
    def _():
        ssum[...] = jnp.zeros((G, H), jnp.float32)
        scnt[...] = jnp.zeros((G, 1), jnp.float32)
        smax[...] = jnp.full((G, H), -jnp.inf, jnp.float32)

    oh = (bb == lax.broadcasted_iota(jnp.int32, (BLK, G), 1))
    ohf = oh.astype(jnp.float32)
    dn = (((0,), (0,)), ((), ()))
    ssum[...] += lax.dot_general(ohf, hb, dn,
                                 preferred_element_type=jnp.float32)
    scnt[...] += lax.dot_general(ohf, jnp.ones((BLK, 1), jnp.float32), dn,
                                 preferred_element_type=jnp.float32)

    g0 = bb[0, 0]
    g1 = bb[BLK - 1, 0]
    for g in range(G):
        @pl.when((g0 <= g) & (g <= g1))
        def _():
            m = jnp.where(bb == g, hb, -jnp.inf)
            mg = jnp.max(m, axis=0, keepdims=True)
            smax[g:g + 1, :] = jnp.maximum(smax[g:g + 1, :], mg)

    @pl.when(b == NBLK - 1)
    def _():
        cnt = jnp.maximum(scnt[...], 1.0)
        mean = ssum[...] / cnt
        sm = smax[...]
        ss = ssum[...]
        ge = jnp.dot(
            jax.nn.relu(jnp.dot(gf_ref[...], wg1_ref[...],
                                preferred_element_type=jnp.float32)
                        + bg1_ref[...]),
            wg2_ref[...], preferred_element_type=jnp.float32) + bg2_ref[...]
        zw = (jnp.dot(mean, wd1_ref[0:H, :],
                      preferred_element_type=jnp.float32)
              + jnp.dot(sm, wd1_ref[H:2 * H, :],
                        preferred_element_type=jnp.float32)
              + jnp.dot(ss, wd1_ref[2 * H:3 * H, :],
                        preferred_element_type=jnp.float32)
              + jnp.dot(ge, wd1_ref[3 * H:4 * H, :],
                        preferred_element_type=jnp.float32)
              + bd1_ref[...])
        d1 = jax.nn.relu(_ln(zw, gd_ref[...], bd_ref[...]))
        d2 = jax.nn.relu(jnp.dot(d1, wd2_ref[...],
                                 preferred_element_type=jnp.float32)
                         + bd2_ref[...])
        lg = jnp.dot(d2, wd3_ref[...],
                     preferred_element_type=jnp.float32) + bd3_ref[...]
        mx = jnp.max(lg, axis=-1, keepdims=True)
        e = jnp.exp(lg - mx)
        out_ref[...] = e / jnp.sum(e, axis=-1, keepdims=True)


def _tc_pool_decode(h, hp, a0, a1, dis, bc, gc, bn, batch2d, gf, p):
    OUT = p['Wd3'].shape[1]
    return pl.pallas_call(
        _pool_body,
        grid=(NBLK,),
        in_specs=[_row_spec(H), _row_spec(H), _row_spec(H), _row_spec(H),
                  _row_spec(1), _const_spec((1, H)), _const_spec((1, H)),
                  _const_spec((1, H)),
                  pl.BlockSpec((BLK, 1), lambda b: (b, 0)),
                  _const_spec((G, 4)),
                  _const_spec((4, H // 2)), _const_spec((1, H // 2)),
                  _const_spec((H // 2, H)), _const_spec((1, H)),
                  _const_spec((4 * H, 2 * H)), _const_spec((1, 2 * H)),
                  _const_spec((1, 2 * H)), _const_spec((1, 2 * H)),
                  _const_spec((2 * H, H)), _const_spec((1, H)),
                  _const_spec((H, OUT)), _const_spec((1, OUT))],
        out_specs=[pl.BlockSpec((G, OUT), lambda b: (0, 0))],
        out_shape=[jax.ShapeDtypeStruct((G, OUT), jnp.float32)],
        scratch_shapes=[pltpu.VMEM((G, H), jnp.float32),
                        pltpu.VMEM((G, 1), jnp.float32),
                        pltpu.VMEM((G, H), jnp.float32)],
    )(h, hp, a0, a1, dis, bc, gc, bn, batch2d,
      gf, p['Wg1'], p['bg1'].reshape(1, -1), p['Wg2'], p['bg2'].reshape(1, -1),
      p['Wd1'], p['bd1'].reshape(1, -1), p['gd'].reshape(1, -1),
      p['bd'].reshape(1, -1), p['Wd2'], p['bd2'].reshape(1, -1),
      p['Wd3'], p['bd3'].reshape(1, -1))[0]


def kernel(x, edge_index, batch, global_features, params):
    p = params
    pad = jnp.arange(EPADN, dtype=jnp.int32)
    src = jnp.concatenate([edge_index[0], pad % N])
    dst = jnp.concatenate([edge_index[1], N + pad % (NPAD - N)])

    degp = _sc_degree(dst)
    d0 = degp[0].reshape(NPAD, 1)
    d1 = degp[1].reshape(NPAD, 1)

    h, hp, dis = _tc_encoder(
        x, d0, d1, p['W_enc'], p['b_enc'].reshape(1, -1),
        p['g_enc'].reshape(1, -1), p['be_enc'].reshape(1, -1), p['Wc'][0])

    for i in range(NL):
        agg = _sc_agg(hp, src, dst)
        bc = p['bc'][i].reshape(1, -1)
        gc = p['gc'][i].reshape(1, -1)
        bn = p['bnc'][i].reshape(1, -1)
        if i < NL - 1:
            h, hp = _tc_layer(h, hp, agg[0], agg[1], dis, bc, gc, bn,
                              p['Wc'][i + 1])
        else:
            return _tc_pool_decode(h, hp, agg[0], agg[1], dis, bc, gc, bn,
                                   batch.reshape(N, 1), global_features, p)


# speedup vs baseline: 24.6246x; 1.0062x over previous
import functools

import jax
import jax.numpy as jnp
from jax import lax
from jax.experimental import pallas as pl
from jax.experimental.pallas import tpu as pltpu
from jax.experimental.pallas import tpu_sc as plsc

N = 10000
E = 320000
H = 128
G = 64
NL = 4

NC = 2
NS = 16
NW = NC * NS
CHUNK = 128
NCHUNK = 80
EW = NCHUNK * CHUNK
EPADN = NW * EW - E
NPAD = 10240
ZROWS = NPAD // NS

_mesh = plsc.VectorSubcoreMesh(core_axis_name="c", subcore_axis_name="s")


def _fill_1d(ref, n, value):
    for j in range(n // 16):
        ref[pl.ds(16 * j, 16)] = jnp.full((16,), value, jnp.float32)


def _sc_degree_body(dst_hbm, out_hbm, didx2, ones, zbuf, deg_acc, sem):
    c = lax.axis_index("c")
    s = lax.axis_index("s")
    wid = s * NC + c

    _fill_1d(ones, CHUNK, 1.0)
    _fill_1d(zbuf, ZROWS, 0.0)
    pltpu.sync_copy(dst_hbm.at[pl.ds(wid * EW, EW)], didx2)
    pltpu.sync_copy(zbuf, deg_acc.at[pl.ds(ZROWS * s, ZROWS)])
    plsc.subcore_barrier()

    FIRE = 8

    def body(k, _):
        for j in range(FIRE):
            i = FIRE * k + j
            pltpu.async_copy(
                ones, deg_acc.at[didx2.at[pl.ds(CHUNK * i, CHUNK)]], sem,
                add=True)
        for j in range(FIRE):
            pltpu.make_async_copy(
                ones, deg_acc.at[didx2.at[pl.ds(0, CHUNK)]], sem).wait()
        return ()

    lax.fori_loop(0, NCHUNK // FIRE, body, (), unroll=False)
    plsc.subcore_barrier()
    pltpu.sync_copy(deg_acc.at[pl.ds(ZROWS * s, ZROWS)],
                    out_hbm.at[c, pl.ds(ZROWS * s, ZROWS)])


_sc_degree = functools.partial(
    pl.kernel,
    out_type=jax.ShapeDtypeStruct((NC, NPAD), jnp.float32),
    mesh=_mesh,
    scratch_types=[
        pltpu.VMEM((EW,), jnp.int32),
        pltpu.VMEM((CHUNK,), jnp.float32),
        pltpu.VMEM((ZROWS,), jnp.float32),
        pltpu.VMEM_SHARED((NPAD,), jnp.float32),
        pltpu.SemaphoreType.DMA,
    ],
)(_sc_degree_body)


def _sc_agg_body(hp_hbm, src_hbm, dst_hbm, out_hbm,
                 didx2, sbufs, rows, zb, acc, isems, gsems):
    c = lax.axis_index("c")
    s = lax.axis_index("s")
    wid = s * NC + c

    for i in range(16):
        for j in range(8):
            zb[i, pl.ds(16 * j, 16)] = jnp.zeros((16,), jnp.float32)

    pltpu.sync_copy(dst_hbm.at[pl.ds(wid * EW, EW)], didx2)

    for k in range(ZROWS // 16):
        pltpu.async_copy(zb, acc.at[pl.ds(ZROWS * s + 16 * k, 16)],
                         gsems[0])
    for k in range(ZROWS // 16):
        pltpu.make_async_copy(zb, acc.at[pl.ds(0, 16)], gsems[0]).wait()
    plsc.subcore_barrier()

    def _ifire(i, q):
        pltpu.async_copy(src_hbm.at[pl.ds(wid * EW + CHUNK * i, CHUNK)],
                         sbufs.at[q], isems[q])

    def _fire(i, q):
        pltpu.make_async_copy(src_hbm.at[pl.ds(0, CHUNK)],
                              sbufs.at[q], isems[q]).wait()
        pltpu.async_copy(hp_hbm.at[sbufs.at[q]], rows.at[q], gsems[q])

    def _gwait(q):
        pltpu.make_async_copy(hp_hbm.at[sbufs.at[0]],
                              rows.at[q], gsems[q]).wait()

    def _scat(i, q):
        pltpu.sync_copy(rows.at[q],
                        acc.at[didx2.at[pl.ds(CHUNK * i, CHUNK)]],
                        add=True)

    _ifire(0, 0)
    _ifire(1, 1)
    _fire(0, 0)

    def body(k, _):
        a = 2 * k
        _fire(a + 1, 1)
        _gwait(0)
        _ifire(a + 2, 0)
        _scat(a, 0)
        _fire(a + 2, 0)
        _gwait(1)
        _ifire(a + 3, 1)
        _scat(a + 1, 1)
        return ()

    lax.fori_loop(0, NCHUNK // 2 - 1, body, (), unroll=False)
    _fire(NCHUNK - 1, 1)
    _gwait(0)
    _scat(NCHUNK - 2, 0)
    _gwait(1)
    _scat(NCHUNK - 1, 1)

    plsc.subcore_barrier()
    pltpu.sync_copy(acc.at[pl.ds(ZROWS * s, ZROWS)],
                    out_hbm.at[c, pl.ds(ZROWS * s, ZROWS)])


_sc_agg = functools.partial(
    pl.kernel,
    out_type=jax.ShapeDtypeStruct((NC, NPAD, H), jnp.float32),
    mesh=_mesh,
    scratch_types=[
        pltpu.VMEM((EW,), jnp.int32),
        pltpu.VMEM((2, CHUNK), jnp.int32),
        pltpu.VMEM((2, CHUNK, H), jnp.float32),
        pltpu.VMEM((16, H), jnp.float32),
        pltpu.VMEM_SHARED((NPAD, H), jnp.float32),
        [pltpu.SemaphoreType.DMA] * 2,
        [pltpu.SemaphoreType.DMA] * 2,
    ],
)(_sc_agg_body)


BLK = 2000
NBLK = N // BLK
EPS = 1e-5


def _ln(h, g, b):
    m = jnp.mean(h, axis=-1, keepdims=True)
    v = jnp.mean((h - m) ** 2, axis=-1, keepdims=True)
    return (h - m) * lax.rsqrt(v + EPS) * g + b


def _enc_body(x_ref, d0_ref, d1_ref, we_ref, be_ref, g_ref, bn_ref, w0_ref,
              h_ref, hp_ref, dis_ref):
    xb = x_ref[...]
    h = jnp.dot(xb, we_ref[...], preferred_element_type=jnp.float32)
    h = h + be_ref[...]
    h = jax.nn.relu(_ln(h, g_ref[...], bn_ref[...]))
    dis = lax.rsqrt(1.0 + d0_ref[...] + d1_ref[...])
    hp = jnp.dot(h, w0_ref[...], preferred_element_type=jnp.float32) * dis
    h_ref[...] = h
    hp_ref[...] = hp
    dis_ref[...] = dis


def _row_spec(w):
    return pl.BlockSpec((BLK, w), lambda b: (b, 0))


def _const_spec(shape):
    return pl.BlockSpec(shape, lambda b: tuple(0 for _ in shape))


def _tc_encoder(x, d0, d1, we, be, g, bn, w0):
    return pl.pallas_call(
        _enc_body,
        grid=(NBLK,),
        in_specs=[_row_spec(H), _row_spec(1), _row_spec(1),
                  _const_spec((H, H)), _const_spec((1, H)),
                  _const_spec((1, H)), _const_spec((1, H)),
                  _const_spec((H, H))],
        out_specs=[_row_spec(H), _row_spec(H), _row_spec(1)],
        out_shape=[jax.ShapeDtypeStruct((N, H), jnp.float32),
                   jax.ShapeDtypeStruct((N, H), jnp.float32),
                   jax.ShapeDtypeStruct((N, 1), jnp.float32)],
    )(x, d0, d1, we, be, g, bn, w0)


def _layer_body(h_ref, hp_ref, a0_ref, a1_ref, dis_ref, bc_ref, gc_ref,
                bn_ref, wn_ref, h2_ref, hp2_ref):
    dis = dis_ref[...]
    o = (a0_ref[...] + a1_ref[...] + hp_ref[...]) * dis + bc_ref[...]
    h2 = jax.nn.relu(_ln(o, gc_ref[...], bn_ref[...])) + h_ref[...]
    hp2 = jnp.dot(h2, wn_ref[...], preferred_element_type=jnp.float32) * dis
    h2_ref[...] = h2
    hp2_ref[...] = hp2


def _tc_layer(h, hp, a0, a1, dis, bc, gc, bn, wn):
    return pl.pallas_call(
        _layer_body,
        grid=(NBLK,),
        in_specs=[_row_spec(H), _row_spec(H), _row_spec(H), _row_spec(H),
                  _row_spec(1), _const_spec((1, H)), _const_spec((1, H)),
                  _const_spec((1, H)), _const_spec((H, H))],
        out_specs=[_row_spec(H), _row_spec(H)],
        out_shape=[jax.ShapeDtypeStruct((N, H), jnp.float32),
                   jax.ShapeDtypeStruct((N, H), jnp.float32)],
    )(h, hp, a0, a1, dis, bc, gc, bn, wn)


def _pool_body(h_ref, hp_ref, a0_ref, a1_ref, dis_ref, bc_ref, gc_ref,
               bn_ref, b_ref, gf_ref, wg1_ref, bg1_ref, wg2_ref, bg2_ref,
               wd1_ref, bd1_ref, gd_ref, bd_ref, wd2_ref, bd2_ref,
               wd3_ref, bd3_ref, out_ref, ssum, scnt, smax):
    b = pl.program_id(0)
    o = (a0_ref[...] + a1_ref[...] + hp_ref[...]) * dis_ref[...] + bc_ref[...]
    hb = jax.nn.relu(_ln(o, gc_ref[...], bn_ref[...])) + h_ref[...]
    bb = b_ref[...]

    @pl.when(b == 0)
    def _():
        ssum[...] = jnp.zeros((G, H), jnp.float32)
        scnt[...] = jnp.zeros((G, 1), jnp.float32)
        smax[...] = jnp.full((G, H), -jnp.inf, jnp.float32)

    oh = (bb == lax.broadcasted_iota(jnp.int32, (BLK, G), 1))
    ohf = oh.astype(jnp.float32)
    dn = (((0,), (0,)), ((), ()))
    ssum[...] += lax.dot_general(ohf, hb, dn,
                                 preferred_element_type=jnp.float32)
    scnt[...] += lax.dot_general(ohf, jnp.ones((BLK, 1), jnp.float32), dn,
                                 preferred_element_type=jnp.float32)

    g0 = bb[0, 0]
    g1 = bb[BLK - 1, 0]
    for g in range(G):
        @pl.when((g0 <= g) & (g <= g1))
        def _():
            m = jnp.where(bb == g, hb, -jnp.inf)
            mg = jnp.max(m, axis=0, keepdims=True)
            smax[g:g + 1, :] = jnp.maximum(smax[g:g + 1, :], mg)

    @pl.when(b == NBLK - 1)
    def _():
        cnt = jnp.maximum(scnt[...], 1.0)
        mean = ssum[...] / cnt
        sm = smax[...]
        ss = ssum[...]
        ge = jnp.dot(
            jax.nn.relu(jnp.dot(gf_ref[...], wg1_ref[...],
                                preferred_element_type=jnp.float32)
                        + bg1_ref[...]),
            wg2_ref[...], preferred_element_type=jnp.float32) + bg2_ref[...]
        zw = (jnp.dot(mean, wd1_ref[0:H, :],
                      preferred_element_type=jnp.float32)
              + jnp.dot(sm, wd1_ref[H:2 * H, :],
                        preferred_element_type=jnp.float32)
              + jnp.dot(ss, wd1_ref[2 * H:3 * H, :],
                        preferred_element_type=jnp.float32)
              + jnp.dot(ge, wd1_ref[3 * H:4 * H, :],
                        preferred_element_type=jnp.float32)
              + bd1_ref[...])
        d1 = jax.nn.relu(_ln(zw, gd_ref[...], bd_ref[...]))
        d2 = jax.nn.relu(jnp.dot(d1, wd2_ref[...],
                                 preferred_element_type=jnp.float32)
                         + bd2_ref[...])
        lg = jnp.dot(d2, wd3_ref[...],
                     preferred_element_type=jnp.float32) + bd3_ref[...]
        mx = jnp.max(lg, axis=-1, keepdims=True)
        e = jnp.exp(lg - mx)
        out_ref[...] = e / jnp.sum(e, axis=-1, keepdims=True)


def _tc_pool_decode(h, hp, a0, a1, dis, bc, gc, bn, batch2d, gf, p):
    OUT = p['Wd3'].shape[1]
    return pl.pallas_call(
        _pool_body,
        grid=(NBLK,),
        in_specs=[_row_spec(H), _row_spec(H), _row_spec(H), _row_spec(H),
                  _row_spec(1), _const_spec((1, H)), _const_spec((1, H)),
                  _const_spec((1, H)),
                  pl.BlockSpec((BLK, 1), lambda b: (b, 0)),
                  _const_spec((G, 4)),
                  _const_spec((4, H // 2)), _const_spec((1, H // 2)),
                  _const_spec((H // 2, H)), _const_spec((1, H)),
                  _const_spec((4 * H, 2 * H)), _const_spec((1, 2 * H)),
                  _const_spec((1, 2 * H)), _const_spec((1, 2 * H)),
                  _const_spec((2 * H, H)), _const_spec((1, H)),
                  _const_spec((H, OUT)), _const_spec((1, OUT))],
        out_specs=[pl.BlockSpec((G, OUT), lambda b: (0, 0))],
        out_shape=[jax.ShapeDtypeStruct((G, OUT), jnp.float32)],
        scratch_shapes=[pltpu.VMEM((G, H), jnp.float32),
                        pltpu.VMEM((G, 1), jnp.float32),
                        pltpu.VMEM((G, H), jnp.float32)],
    )(h, hp, a0, a1, dis, bc, gc, bn, batch2d,
      gf, p['Wg1'], p['bg1'].reshape(1, -1), p['Wg2'], p['bg2'].reshape(1, -1),
      p['Wd1'], p['bd1'].reshape(1, -1), p['gd'].reshape(1, -1),
      p['bd'].reshape(1, -1), p['Wd2'], p['bd2'].reshape(1, -1),
      p['Wd3'], p['bd3'].reshape(1, -1))[0]


def kernel(x, edge_index, batch, global_features, params):
    p = params
    pad = jnp.arange(EPADN, dtype=jnp.int32)
    src = jnp.concatenate([edge_index[0], pad % N])
    dst = jnp.concatenate([edge_index[1], N + pad % (NPAD - N)])

    degp = _sc_degree(dst)
    d0 = degp[0].reshape(NPAD, 1)
    d1 = degp[1].reshape(NPAD, 1)

    h, hp, dis = _tc_encoder(
        x, d0, d1, p['W_enc'], p['b_enc'].reshape(1, -1),
        p['g_enc'].reshape(1, -1), p['be_enc'].reshape(1, -1), p['Wc'][0])

    for i in range(NL):
        agg = _sc_agg(hp, src, dst)
        bc = p['bc'][i].reshape(1, -1)
        gc = p['gc'][i].reshape(1, -1)
        bn = p['bnc'][i].reshape(1, -1)
        if i < NL - 1:
            h, hp = _tc_layer(h, hp, agg[0], agg[1], dis, bc, gc, bn,
                              p['Wc'][i + 1])
        else:
            return _tc_pool_decode(h, hp, agg[0], agg[1], dis, bc, gc, bn,
                                   batch.reshape(N, 1), global_features, p)


# speedup vs baseline: 24.8950x; 1.0110x over previous
import functools

import jax
import jax.numpy as jnp
from jax import lax
from jax.experimental import pallas as pl
from jax.experimental.pallas import tpu as pltpu
from jax.experimental.pallas import tpu_sc as plsc

N = 10000
E = 320000
H = 128
G = 64
NL = 4

NC = 2
NS = 16
NW = NC * NS
CHUNK = 128
NCHUNK = 80
EW = NCHUNK * CHUNK
EPADN = NW * EW - E
NPAD = 10240
ZROWS = NPAD // NS

_mesh = plsc.VectorSubcoreMesh(core_axis_name="c", subcore_axis_name="s")


def _fill_1d(ref, n, value):
    for j in range(n // 16):
        ref[pl.ds(16 * j, 16)] = jnp.full((16,), value, jnp.float32)


def _sc_degree_body(dst_hbm, out_hbm, didx2, ones, zbuf, deg_acc, sem):
    c = lax.axis_index("c")
    s = lax.axis_index("s")
    wid = s * NC + c

    _fill_1d(ones, CHUNK, 1.0)
    _fill_1d(zbuf, ZROWS, 0.0)
    pltpu.sync_copy(dst_hbm.at[pl.ds(wid * EW, EW)], didx2)
    pltpu.sync_copy(zbuf, deg_acc.at[pl.ds(ZROWS * s, ZROWS)])
    plsc.subcore_barrier()

    FIRE = 8

    def body(k, _):
        for j in range(FIRE):
            i = FIRE * k + j
            pltpu.async_copy(
                ones, deg_acc.at[didx2.at[pl.ds(CHUNK * i, CHUNK)]], sem,
                add=True)
        for j in range(FIRE):
            pltpu.make_async_copy(
                ones, deg_acc.at[didx2.at[pl.ds(0, CHUNK)]], sem).wait()
        return ()

    lax.fori_loop(0, NCHUNK // FIRE, body, (), unroll=False)
    plsc.subcore_barrier()
    pltpu.sync_copy(deg_acc.at[pl.ds(ZROWS * s, ZROWS)],
                    out_hbm.at[c, pl.ds(ZROWS * s, ZROWS)])


_sc_degree = functools.partial(
    pl.kernel,
    out_type=jax.ShapeDtypeStruct((NC, NPAD), jnp.float32),
    mesh=_mesh,
    scratch_types=[
        pltpu.VMEM((EW,), jnp.int32),
        pltpu.VMEM((CHUNK,), jnp.float32),
        pltpu.VMEM((ZROWS,), jnp.float32),
        pltpu.VMEM_SHARED((NPAD,), jnp.float32),
        pltpu.SemaphoreType.DMA,
    ],
)(_sc_degree_body)


def _sc_agg_body(hp_hbm, src_hbm, dst_hbm, out_hbm,
                 didx2, sbufs, rows, zb, acc, isems, gsems, zsem):
    c = lax.axis_index("c")
    s = lax.axis_index("s")
    wid = s * NC + c

    for i in range(16):
        for j in range(8):
            zb[i, pl.ds(16 * j, 16)] = jnp.zeros((16,), jnp.float32)

    pltpu.sync_copy(dst_hbm.at[pl.ds(wid * EW, EW)], didx2)

    for k in range(ZROWS // 16):
        pltpu.async_copy(zb, acc.at[pl.ds(ZROWS * s + 16 * k, 16)],
                         zsem)

    def _ifire(i, q):
        pltpu.async_copy(src_hbm.at[pl.ds(wid * EW + CHUNK * i, CHUNK)],
                         sbufs.at[q], isems[q])

    def _fire(i, q):
        pltpu.make_async_copy(src_hbm.at[pl.ds(0, CHUNK)],
                              sbufs.at[q], isems[q]).wait()
        pltpu.async_copy(hp_hbm.at[sbufs.at[q]], rows.at[q], gsems[q])

    def _gwait(q):
        pltpu.make_async_copy(hp_hbm.at[sbufs.at[0]],
                              rows.at[q], gsems[q]).wait()

    def _scat(i, q):
        pltpu.sync_copy(rows.at[q],
                        acc.at[didx2.at[pl.ds(CHUNK * i, CHUNK)]],
                        add=True)

    _ifire(0, 0)
    _ifire(1, 1)
    _fire(0, 0)
    for k in range(ZROWS // 16):
        pltpu.make_async_copy(zb, acc.at[pl.ds(0, 16)], zsem).wait()
    plsc.subcore_barrier()

    def body(k, _):
        a = 2 * k
        _fire(a + 1, 1)
        _gwait(0)
        _ifire(a + 2, 0)
        _scat(a, 0)
        _fire(a + 2, 0)
        _gwait(1)
        _ifire(a + 3, 1)
        _scat(a + 1, 1)
        return ()

    lax.fori_loop(0, NCHUNK // 2 - 1, body, (), unroll=False)
    _fire(NCHUNK - 1, 1)
    _gwait(0)
    _scat(NCHUNK - 2, 0)
    _gwait(1)
    _scat(NCHUNK - 1, 1)

    plsc.subcore_barrier()
    pltpu.sync_copy(acc.at[pl.ds(ZROWS * s, ZROWS)],
                    out_hbm.at[c, pl.ds(ZROWS * s, ZROWS)])


_sc_agg = functools.partial(
    pl.kernel,
    out_type=jax.ShapeDtypeStruct((NC, NPAD, H), jnp.float32),
    mesh=_mesh,
    scratch_types=[
        pltpu.VMEM((EW,), jnp.int32),
        pltpu.VMEM((2, CHUNK), jnp.int32),
        pltpu.VMEM((2, CHUNK, H), jnp.float32),
        pltpu.VMEM((16, H), jnp.float32),
        pltpu.VMEM_SHARED((NPAD, H), jnp.float32),
        [pltpu.SemaphoreType.DMA] * 2,
        [pltpu.SemaphoreType.DMA] * 2,
        pltpu.SemaphoreType.DMA,
    ],
)(_sc_agg_body)


BLK = 2000
NBLK = N // BLK
EPS = 1e-5


def _ln(h, g, b):
    m = jnp.mean(h, axis=-1, keepdims=True)
    v = jnp.mean((h - m) ** 2, axis=-1, keepdims=True)
    return (h - m) * lax.rsqrt(v + EPS) * g + b


def _enc_body(x_ref, d0_ref, d1_ref, we_ref, be_ref, g_ref, bn_ref, w0_ref,
              h_ref, hp_ref, dis_ref):
    xb = x_ref[...]
    h = jnp.dot(xb, we_ref[...], preferred_element_type=jnp.float32)
    h = h + be_ref[...]
    h = jax.nn.relu(_ln(h, g_ref[...], bn_ref[...]))
    dis = lax.rsqrt(1.0 + d0_ref[...] + d1_ref[...])
    hp = jnp.dot(h, w0_ref[...], preferred_element_type=jnp.float32) * dis
    h_ref[...] = h
    hp_ref[...] = hp
    dis_ref[...] = dis


def _row_spec(w):
    return pl.BlockSpec((BLK, w), lambda b: (b, 0))


def _const_spec(shape):
    return pl.BlockSpec(shape, lambda b: tuple(0 for _ in shape))


def _tc_encoder(x, d0, d1, we, be, g, bn, w0):
    return pl.pallas_call(
        _enc_body,
        grid=(NBLK,),
        in_specs=[_row_spec(H), _row_spec(1), _row_spec(1),
                  _const_spec((H, H)), _const_spec((1, H)),
                  _const_spec((1, H)), _const_spec((1, H)),
                  _const_spec((H, H))],
        out_specs=[_row_spec(H), _row_spec(H), _row_spec(1)],
        out_shape=[jax.ShapeDtypeStruct((N, H), jnp.float32),
                   jax.ShapeDtypeStruct((N, H), jnp.float32),
                   jax.ShapeDtypeStruct((N, 1), jnp.float32)],
    )(x, d0, d1, we, be, g, bn, w0)


def _layer_body(h_ref, hp_ref, a0_ref, a1_ref, dis_ref, bc_ref, gc_ref,
                bn_ref, wn_ref, h2_ref, hp2_ref):
    dis = dis_ref[...]
    o = (a0_ref[...] + a1_ref[...] + hp_ref[...]) * dis + bc_ref[...]
    h2 = jax.nn.relu(_ln(o, gc_ref[...], bn_ref[...])) + h_ref[...]
    hp2 = jnp.dot(h2, wn_ref[...], preferred_element_type=jnp.float32) * dis
    h2_ref[...] = h2
    hp2_ref[...] = hp2


def _tc_layer(h, hp, a0, a1, dis, bc, gc, bn, wn):
    return pl.pallas_call(
        _layer_body,
        grid=(NBLK,),
        in_specs=[_row_spec(H), _row_spec(H), _row_spec(H), _row_spec(H),
                  _row_spec(1), _const_spec((1, H)), _const_spec((1, H)),
                  _const_spec((1, H)), _const_spec((H, H))],
        out_specs=[_row_spec(H), _row_spec(H)],
        out_shape=[jax.ShapeDtypeStruct((N, H), jnp.float32),
                   jax.ShapeDtypeStruct((N, H), jnp.float32)],
    )(h, hp, a0, a1, dis, bc, gc, bn, wn)


def _pool_body(h_ref, hp_ref, a0_ref, a1_ref, dis_ref, bc_ref, gc_ref,
               bn_ref, b_ref, gf_ref, wg1_ref, bg1_ref, wg2_ref, bg2_ref,
               wd1_ref, bd1_ref, gd_ref, bd_ref, wd2_ref, bd2_ref,
               wd3_ref, bd3_ref, out_ref, ssum, scnt, smax):
    b = pl.program_id(0)
    o = (a0_ref[...] + a1_ref[...] + hp_ref[...]) * dis_ref[...] + bc_ref[...]
    hb = jax.nn.relu(_ln(o, gc_ref[...], bn_ref[...])) + h_ref[...]
    bb = b_ref[...]

    @pl.when(b == 0)
    def _():
        ssum[...] = jnp.zeros((G, H), jnp.float32)
        scnt[...] = jnp.zeros((G, 1), jnp.float32)
        smax[...] = jnp.full((G, H), -jnp.inf, jnp.float32)

    oh = (bb == lax.broadcasted_iota(jnp.int32, (BLK, G), 1))
    ohf = oh.astype(jnp.float32)
    dn = (((0,), (0,)), ((), ()))
    ssum[...] += lax.dot_general(ohf, hb, dn,
                                 preferred_element_type=jnp.float32)
    scnt[...] += lax.dot_general(ohf, jnp.ones((BLK, 1), jnp.float32), dn,
                                 preferred_element_type=jnp.float32)

    g0 = bb[0, 0]
    g1 = bb[BLK - 1, 0]
    for g in range(G):
        @pl.when((g0 <= g) & (g <= g1))
        def _():
            m = jnp.where(bb == g, hb, -jnp.inf)
            mg = jnp.max(m, axis=0, keepdims=True)
            smax[g:g + 1, :] = jnp.maximum(smax[g:g + 1, :], mg)

    @pl.when(b == NBLK - 1)
    def _():
        cnt = jnp.maximum(scnt[...], 1.0)
        mean = ssum[...] / cnt
        sm = smax[...]
        ss = ssum[...]
        ge = jnp.dot(
            jax.nn.relu(jnp.dot(gf_ref[...], wg1_ref[...],
                                preferred_element_type=jnp.float32)
                        + bg1_ref[...]),
            wg2_ref[...], preferred_element_type=jnp.float32) + bg2_ref[...]
        zw = (jnp.dot(mean, wd1_ref[0:H, :],
                      preferred_element_type=jnp.float32)
              + jnp.dot(sm, wd1_ref[H:2 * H, :],
                        preferred_element_type=jnp.float32)
              + jnp.dot(ss, wd1_ref[2 * H:3 * H, :],
                        preferred_element_type=jnp.float32)
              + jnp.dot(ge, wd1_ref[3 * H:4 * H, :],
                        preferred_element_type=jnp.float32)
              + bd1_ref[...])
        d1 = jax.nn.relu(_ln(zw, gd_ref[...], bd_ref[...]))
        d2 = jax.nn.relu(jnp.dot(d1, wd2_ref[...],
                                 preferred_element_type=jnp.float32)
                         + bd2_ref[...])
        lg = jnp.dot(d2, wd3_ref[...],
                     preferred_element_type=jnp.float32) + bd3_ref[...]
        mx = jnp.max(lg, axis=-1, keepdims=True)
        e = jnp.exp(lg - mx)
        out_ref[...] = e / jnp.sum(e, axis=-1, keepdims=True)


def _tc_pool_decode(h, hp, a0, a1, dis, bc, gc, bn, batch2d, gf, p):
    OUT = p['Wd3'].shape[1]
    return pl.pallas_call(
        _pool_body,
        grid=(NBLK,),
        in_specs=[_row_spec(H), _row_spec(H), _row_spec(H), _row_spec(H),
                  _row_spec(1), _const_spec((1, H)), _const_spec((1, H)),
                  _const_spec((1, H)),
                  pl.BlockSpec((BLK, 1), lambda b: (b, 0)),
                  _const_spec((G, 4)),
                  _const_spec((4, H // 2)), _const_spec((1, H // 2)),
                  _const_spec((H // 2, H)), _const_spec((1, H)),
                  _const_spec((4 * H, 2 * H)), _const_spec((1, 2 * H)),
                  _const_spec((1, 2 * H)), _const_spec((1, 2 * H)),
                  _const_spec((2 * H, H)), _const_spec((1, H)),
                  _const_spec((H, OUT)), _const_spec((1, OUT))],
        out_specs=[pl.BlockSpec((G, OUT), lambda b: (0, 0))],
        out_shape=[jax.ShapeDtypeStruct((G, OUT), jnp.float32)],
        scratch_shapes=[pltpu.VMEM((G, H), jnp.float32),
                        pltpu.VMEM((G, 1), jnp.float32),
                        pltpu.VMEM((G, H), jnp.float32)],
    )(h, hp, a0, a1, dis, bc, gc, bn, batch2d,
      gf, p['Wg1'], p['bg1'].reshape(1, -1), p['Wg2'], p['bg2'].reshape(1, -1),
      p['Wd1'], p['bd1'].reshape(1, -1), p['gd'].reshape(1, -1),
      p['bd'].reshape(1, -1), p['Wd2'], p['bd2'].reshape(1, -1),
      p['Wd3'], p['bd3'].reshape(1, -1))[0]


def kernel(x, edge_index, batch, global_features, params):
    p = params
    pad = jnp.arange(EPADN, dtype=jnp.int32)
    src = jnp.concatenate([edge_index[0], pad % N])
    dst = jnp.concatenate([edge_index[1], N + pad % (NPAD - N)])

    degp = _sc_degree(dst)
    d0 = degp[0].reshape(NPAD, 1)
    d1 = degp[1].reshape(NPAD, 1)

    h, hp, dis = _tc_encoder(
        x, d0, d1, p['W_enc'], p['b_enc'].reshape(1, -1),
        p['g_enc'].reshape(1, -1), p['be_enc'].reshape(1, -1), p['Wc'][0])

    for i in range(NL):
        agg = _sc_agg(hp, src, dst)
        bc = p['bc'][i].reshape(1, -1)
        gc = p['gc'][i].reshape(1, -1)
        bn = p['bnc'][i].reshape(1, -1)
        if i < NL - 1:
            h, hp = _tc_layer(h, hp, agg[0], agg[1], dis, bc, gc, bn,
                              p['Wc'][i + 1])
        else:
            return _tc_pool_decode(h, hp, agg[0], agg[1], dis, bc, gc, bn,
                                   batch.reshape(N, 1), global_features, p)
